# Initial kernel scaffold; baseline (speedup 1.0000x reference)
import jax
import jax.numpy as jnp
from jax.experimental import pallas as pl


def kernel(x, ids):
    raise NotImplementedError("write your pallas kernel here")


# speedup vs baseline: 1.0614x; 1.0614x over previous
import functools

import jax
import jax.numpy as jnp
from jax import lax
from jax.experimental import pallas as pl
from jax.experimental.pallas import tpu as pltpu
from jax.experimental.pallas import tpu_sc as plsc

R, C, V = 16384, 128, 100000
NC, NS, L = 2, 16, 16
NW = NC * NS
ROWS_PER_W = R // NW
CHUNK_ROWS = 128
NCHUNK = ROWS_PER_W // CHUNK_ROWS

_MESH = plsc.VectorSubcoreMesh(
    core_axis_name="c", subcore_axis_name="s", num_cores=NC, num_subcores=NS
)


def _body(x_hbm, ids_hbm, out_hbm, idx_v, gat_v, sem):
    wid = lax.axis_index("s") * NC + lax.axis_index("c")
    iota = lax.iota(jnp.int32, 16)
    jvecs = [iota + (u * L) for u in range(C // L)]
    row0 = wid * ROWS_PER_W

    for ci in range(NCHUNK):
        rb = row0 + ci * CHUNK_ROWS
        pltpu.sync_copy(ids_hbm.at[pl.ds(rb, CHUNK_ROWS)], idx_v)

        def fixup(r, carry):
            for u in range(C // L):
                sl = (r, pl.ds(u * L, L))
                idx_v[sl] = idx_v[sl] * C + jvecs[u]
            return carry

        lax.fori_loop(0, CHUNK_ROWS, fixup, 0)

        def fire(r, carry):
            pltpu.async_copy(x_hbm.at[idx_v.at[r]], gat_v.at[r], sem)
            return carry

        lax.fori_loop(0, CHUNK_ROWS, fire, 0)
        pltpu.make_async_copy(
            out_hbm.at[pl.ds(rb, CHUNK_ROWS)], gat_v, sem
        ).wait()
        pltpu.sync_copy(gat_v, out_hbm.at[pl.ds(rb, CHUNK_ROWS)])


@functools.partial(
    pl.kernel,
    out_type=jax.ShapeDtypeStruct((R, C), jnp.float32),
    mesh=_MESH,
    scratch_types=[
        pltpu.VMEM((CHUNK_ROWS, C), jnp.int32),
        pltpu.VMEM((CHUNK_ROWS, C), jnp.float32),
        pltpu.SemaphoreType.DMA,
    ],
)
def _gather_sc(x_flat, ids, out, idx_v, gat_v, sem):
    _body(x_flat, ids, out, idx_v, gat_v, sem)


def kernel(x, ids):
    return _gather_sc(x.reshape(-1), ids.astype(jnp.int32))


# speedup vs baseline: 1.1370x; 1.0712x over previous
import functools

import jax
import jax.numpy as jnp
from jax import lax
from jax.experimental import pallas as pl
from jax.experimental.pallas import tpu as pltpu
from jax.experimental.pallas import tpu_sc as plsc

R, C, V = 16384, 128, 100000
NC, NS, L = 2, 16, 16
NW = NC * NS
ROWS_PER_W = R // NW
CHUNK_ROWS = 128
NCHUNK = ROWS_PER_W // CHUNK_ROWS

_MESH = plsc.VectorSubcoreMesh(
    core_axis_name="c", subcore_axis_name="s", num_cores=NC, num_subcores=NS
)


def _body(x_hbm, ids_hbm, out_hbm, ids_v, fidx_v, gat_v, sem_i, sem_g):
    wid = lax.axis_index("s") * NC + lax.axis_index("c")
    iota = lax.iota(jnp.int32, 16)
    jvecs = [iota + (u * L) for u in range(C // L)]
    row0 = wid * ROWS_PER_W

    def chunk_base(ci):
        return row0 + ci * CHUNK_ROWS

    def load_ids(ci, b):
        pltpu.async_copy(
            ids_hbm.at[pl.ds(chunk_base(ci), CHUNK_ROWS)], ids_v[b], sem_i[b]
        )

    def fix_and_fire(b):
        def body(r, carry):
            for u in range(C // L):
                sl = (r, pl.ds(u * L, L))
                fidx_v[b][sl] = ids_v[b][sl] * C + jvecs[u]
            pltpu.async_copy(x_hbm.at[fidx_v[b].at[r]], gat_v[b].at[r], sem_g[b])
            return carry

        lax.fori_loop(0, CHUNK_ROWS, body, 0)

    def drain_and_store(ci, b):
        pltpu.make_async_copy(
            out_hbm.at[pl.ds(chunk_base(ci), CHUNK_ROWS)], gat_v[b], sem_g[b]
        ).wait()
        pltpu.sync_copy(gat_v[b], out_hbm.at[pl.ds(chunk_base(ci), CHUNK_ROWS)])

    load_ids(0, 0)
    for ci in range(NCHUNK):
        b = ci % 2
        pltpu.make_async_copy(
            ids_hbm.at[pl.ds(chunk_base(ci), CHUNK_ROWS)], ids_v[b], sem_i[b]
        ).wait()
        if ci + 1 < NCHUNK:
            load_ids(ci + 1, 1 - b)
        fix_and_fire(b)
        if ci > 0:
            drain_and_store(ci - 1, 1 - b)
    drain_and_store(NCHUNK - 1, (NCHUNK - 1) % 2)


@functools.partial(
    pl.kernel,
    out_type=jax.ShapeDtypeStruct((R, C), jnp.float32),
    mesh=_MESH,
    scratch_types=[
        [pltpu.VMEM((CHUNK_ROWS, C), jnp.int32) for _ in range(2)],
        [pltpu.VMEM((CHUNK_ROWS, C), jnp.int32) for _ in range(2)],
        [pltpu.VMEM((CHUNK_ROWS, C), jnp.float32) for _ in range(2)],
        [pltpu.SemaphoreType.DMA for _ in range(2)],
        [pltpu.SemaphoreType.DMA for _ in range(2)],
    ],
)
def _gather_sc(x_flat, ids, out, ids_v, fidx_v, gat_v, sem_i, sem_g):
    _body(x_flat, ids, out, ids_v, fidx_v, gat_v, sem_i, sem_g)


def kernel(x, ids):
    return _gather_sc(x.reshape(-1), ids.astype(jnp.int32))
